# Initial kernel scaffold; baseline (speedup 1.0000x reference)
import jax
import jax.numpy as jnp
from jax.experimental import pallas as pl


def kernel(x, edge_index, weights_matrix, W1, b1, W2, b2, Wlin, blin):
    raise NotImplementedError("write your pallas kernel here")


# speedup vs baseline: 7.9041x; 7.9041x over previous
import functools

import jax
import jax.numpy as jnp
from jax import lax
from jax.experimental import pallas as pl
from jax.experimental.pallas import tpu as pltpu
from jax.experimental.pallas import tpu_sc as plsc

N = 10000
E = 320000
D = 128
NC, NS, LANES = 2, 16, 16
NW = NC * NS
CHUNK = 128
NCHUNKS = E // CHUNK
BASE_CH = NCHUNKS // NW
EXTRA_CH = NCHUNKS - BASE_CH * NW
N_PAD = 10240
ROWS_PER_TILE = N_PAD // NS
ZBLK = 128

_mesh = plsc.VectorSubcoreMesh(core_axis_name="c", subcore_axis_name="s")


def _tile_chunk_range():
    c = lax.axis_index("c")
    s = lax.axis_index("s")
    wid = s * NC + c
    start = BASE_CH * wid + jnp.minimum(wid, EXTRA_CH)
    count = BASE_CH + (wid < EXTRA_CH).astype(jnp.int32)
    return c, s, start, count


@functools.partial(
    pl.kernel,
    out_type=jax.ShapeDtypeStruct((NC, N_PAD, LANES), jnp.float32),
    mesh=_mesh,
    scratch_types=[
        pltpu.VMEM_SHARED((N_PAD, LANES), jnp.float32),
        pltpu.VMEM((CHUNK,), jnp.int32),
        pltpu.VMEM((CHUNK, LANES), jnp.float32),
    ],
)
def _sc_degree(dst_hbm, ewb_hbm, out_hbm, acc_sh, dst_v, ewb_v):
    c, s, start, count = _tile_chunk_range()
    row0 = s * ROWS_PER_TILE

    @pl.loop(0, CHUNK)
    def _(r):
        ewb_v[r, :] = jnp.zeros((LANES,), jnp.float32)

    @pl.loop(0, ROWS_PER_TILE // ZBLK)
    def _(i):
        pltpu.sync_copy(ewb_v.at[pl.ds(0, ZBLK)],
                        acc_sh.at[pl.ds(row0 + i * ZBLK, ZBLK)])

    plsc.subcore_barrier()

    @pl.loop(start, start + count)
    def _(ci):
        base = ci * CHUNK
        pltpu.sync_copy(dst_hbm.at[pl.ds(base, CHUNK)], dst_v)
        pltpu.sync_copy(ewb_hbm.at[pl.ds(base, CHUNK)], ewb_v)
        pltpu.sync_copy(ewb_v, acc_sh.at[dst_v], add=True)

    plsc.subcore_barrier()

    @pl.loop(0, ROWS_PER_TILE // ZBLK)
    def _(i):
        r = row0 + i * ZBLK
        pltpu.sync_copy(acc_sh.at[pl.ds(r, ZBLK)], ewb_v.at[pl.ds(0, ZBLK)])
        pltpu.sync_copy(ewb_v.at[pl.ds(0, ZBLK)], out_hbm.at[c, pl.ds(r, ZBLK)])


@functools.partial(
    pl.kernel,
    out_type=jax.ShapeDtypeStruct((NC, N_PAD, D), jnp.float32),
    mesh=_mesh,
    scratch_types=[
        pltpu.VMEM_SHARED((N_PAD, D), jnp.float32),
        pltpu.VMEM((CHUNK,), jnp.int32),
        pltpu.VMEM((CHUNK,), jnp.int32),
        pltpu.VMEM((CHUNK, LANES), jnp.float32),
        pltpu.VMEM((CHUNK, D), jnp.float32),
    ],
)
def _sc_aggregate(src_hbm, dst_hbm, ewb_hbm, hs_hbm, out_hbm,
                  acc_sh, src_v, dst_v, ewb_v, rows_v):
    c, s, start, count = _tile_chunk_range()
    row0 = s * ROWS_PER_TILE

    @pl.loop(0, CHUNK)
    def _(r):
        for k in range(D // LANES):
            rows_v[r, pl.ds(k * LANES, LANES)] = jnp.zeros((LANES,), jnp.float32)

    @pl.loop(0, ROWS_PER_TILE // ZBLK)
    def _(i):
        pltpu.sync_copy(rows_v.at[pl.ds(0, ZBLK)],
                        acc_sh.at[pl.ds(row0 + i * ZBLK, ZBLK)])

    plsc.subcore_barrier()

    @pl.loop(start, start + count)
    def _(ci):
        base = ci * CHUNK
        pltpu.sync_copy(src_hbm.at[pl.ds(base, CHUNK)], src_v)
        pltpu.sync_copy(dst_hbm.at[pl.ds(base, CHUNK)], dst_v)
        pltpu.sync_copy(ewb_hbm.at[pl.ds(base, CHUNK)], ewb_v)
        pltpu.sync_copy(hs_hbm.at[src_v], rows_v)

        @pl.loop(0, CHUNK)
        def _(e):
            w = ewb_v[e, :]
            for k in range(D // LANES):
                sl = pl.ds(k * LANES, LANES)
                rows_v[e, sl] = rows_v[e, sl] * w

        pltpu.sync_copy(rows_v, acc_sh.at[dst_v], add=True)

    plsc.subcore_barrier()

    @pl.loop(0, ROWS_PER_TILE // ZBLK)
    def _(i):
        r = row0 + i * ZBLK
        pltpu.sync_copy(acc_sh.at[pl.ds(r, ZBLK)], rows_v.at[pl.ds(0, ZBLK)])
        pltpu.sync_copy(rows_v.at[pl.ds(0, ZBLK)], out_hbm.at[c, pl.ds(r, ZBLK)])


MBLK = 1000
EBLK = 20000


def _ewb_body(ew_ref, out_ref):
    out_ref[...] = jnp.broadcast_to(ew_ref[...], (EBLK, LANES))


def _ew_broadcast(ew):
    return pl.pallas_call(
        _ewb_body,
        grid=(E // EBLK,),
        in_specs=[pl.BlockSpec((EBLK, 1), lambda i: (i, 0))],
        out_specs=pl.BlockSpec((EBLK, LANES), lambda i: (i, 0)),
        out_shape=jax.ShapeDtypeStruct((E, LANES), jnp.float32),
    )(ew)


def _mm_body(x_ref, w_ref, out_ref):
    out_ref[...] = jnp.dot(x_ref[...], w_ref[...],
                           preferred_element_type=jnp.float32)


def _matmul(x, w):
    return pl.pallas_call(
        _mm_body,
        grid=(N // MBLK,),
        in_specs=[pl.BlockSpec((MBLK, D), lambda i: (i, 0)),
                  pl.BlockSpec((D, D), lambda i: (0, 0))],
        out_specs=pl.BlockSpec((MBLK, D), lambda i: (i, 0)),
        out_shape=jax.ShapeDtypeStruct((N, D), jnp.float32),
    )(x, w)


def _scale_body(dgp_ref, h_ref, hs_ref, dinv_ref):
    deg = dgp_ref[0, :, 0:1] + dgp_ref[1, :, 0:1] + 1.0
    dinv = lax.rsqrt(deg)
    dinv_ref[...] = dinv
    hs_ref[...] = h_ref[...] * dinv


def _scale(dgp, h):
    return pl.pallas_call(
        _scale_body,
        grid=(N // MBLK,),
        in_specs=[pl.BlockSpec((NC, MBLK, LANES), lambda i: (0, i, 0)),
                  pl.BlockSpec((MBLK, D), lambda i: (i, 0))],
        out_specs=[pl.BlockSpec((MBLK, D), lambda i: (i, 0)),
                   pl.BlockSpec((MBLK, 1), lambda i: (i, 0))],
        out_shape=[jax.ShapeDtypeStruct((N, D), jnp.float32),
                   jax.ShapeDtypeStruct((N, 1), jnp.float32)],
    )(dgp, h)


def _elu(t):
    return jnp.where(t > 0.0, t, jnp.exp(t) - 1.0)


def _mid_body(p_ref, hs_ref, dinv_ref, w_ref, b_ref, out_ref):
    t = (p_ref[0] + p_ref[1] + hs_ref[...]) * dinv_ref[...] + b_ref[...]
    a = _elu(t)
    out_ref[...] = jnp.dot(a, w_ref[...],
                           preferred_element_type=jnp.float32) * dinv_ref[...]


def _mid(p, hs, dinv, w, b):
    return pl.pallas_call(
        _mid_body,
        grid=(N // MBLK,),
        in_specs=[pl.BlockSpec((NC, MBLK, D), lambda i: (0, i, 0)),
                  pl.BlockSpec((MBLK, D), lambda i: (i, 0)),
                  pl.BlockSpec((MBLK, 1), lambda i: (i, 0)),
                  pl.BlockSpec((D, D), lambda i: (0, 0)),
                  pl.BlockSpec((1, D), lambda i: (0, 0))],
        out_specs=pl.BlockSpec((MBLK, D), lambda i: (i, 0)),
        out_shape=jax.ShapeDtypeStruct((N, D), jnp.float32),
    )(p, hs, dinv, w, b)


def _final_body(q_ref, hs_ref, dinv_ref, b_ref, wl_ref, bl_ref, out_ref):
    t = (q_ref[0] + q_ref[1] + hs_ref[...]) * dinv_ref[...] + b_ref[...]
    a = _elu(t)
    out_ref[...] = jnp.sum(a * wl_ref[...], axis=1, keepdims=True) + bl_ref[...]


def _final(q, hs, dinv, b, wlin_t, blin):
    return pl.pallas_call(
        _final_body,
        grid=(N // MBLK,),
        in_specs=[pl.BlockSpec((NC, MBLK, D), lambda i: (0, i, 0)),
                  pl.BlockSpec((MBLK, D), lambda i: (i, 0)),
                  pl.BlockSpec((MBLK, 1), lambda i: (i, 0)),
                  pl.BlockSpec((1, D), lambda i: (0, 0)),
                  pl.BlockSpec((1, D), lambda i: (0, 0)),
                  pl.BlockSpec((1, 1), lambda i: (0, 0))],
        out_specs=pl.BlockSpec((MBLK, 1), lambda i: (i, 0)),
        out_shape=jax.ShapeDtypeStruct((N, 1), jnp.float32),
    )(q, hs, dinv, b, wlin_t, blin)


def kernel(x, edge_index, weights_matrix, W1, b1, W2, b2, Wlin, blin):
    src = edge_index[0]
    dst = edge_index[1]

    ewb = _ew_broadcast(weights_matrix.reshape(E, 1))
    dgp = _sc_degree(dst, ewb)
    h1 = _matmul(x, W1)

    hs1, dinv = _scale(dgp, h1)
    p = _sc_aggregate(src, dst, ewb, hs1)
    hs2 = _mid(p, hs1, dinv, W2, b1.reshape(1, D))
    q = _sc_aggregate(src, dst, ewb, hs2)
    out = _final(q, hs2, dinv, b2.reshape(1, D), Wlin.reshape(1, D),
                 blin.reshape(1, 1))
    return out.reshape(N)
